# baseline (speedup 1.0000x reference)
import jax
import jax.numpy as jnp
from jax import lax
from jax.experimental import pallas as pl
from jax.experimental.pallas import tpu as pltpu

_BA = 20480


def _focal_body(cls_ref, reg_ref, anch_ref, ann_ref, nvalid_ref, out_ref):
    i = pl.program_id(1)

    x = jnp.clip(cls_ref[0], 1e-4, 1.0 - 1e-4)
    C, BA = x.shape
    nvalid = nvalid_ref[0]
    valid = (lax.broadcasted_iota(jnp.int32, (1, BA), 1) + i * BA) < nvalid
    x = jnp.where(valid, x, 0.5)
    annb = ann_ref[0]
    M = annb.shape[0]
    bx1 = annb[:, 0:1]
    by1 = annb[:, 1:2]
    bx2 = annb[:, 2:3]
    by2 = annb[:, 3:4]

    ax1 = jnp.where(valid, anch_ref[0:1, :], 0.0)
    ay1 = jnp.where(valid, anch_ref[1:2, :], 0.0)
    ax2 = jnp.where(valid, anch_ref[2:3, :], 16.0)
    ay2 = jnp.where(valid, anch_ref[3:4, :], 16.0)
    aw = ax2 - ax1
    ah = ay2 - ay1
    acx = ax1 + 0.5 * aw
    acy = ay1 + 0.5 * ah
    aw_s = jnp.maximum(aw, 1e-3)
    ah_s = jnp.maximum(ah, 1e-3)

    area_a = aw * ah
    area_b = (bx2 - bx1) * (by2 - by1)
    iw = jnp.maximum(jnp.minimum(ax2, bx2) - jnp.maximum(ax1, bx1), 0.0)
    ih = jnp.maximum(jnp.minimum(ay2, by2) - jnp.maximum(ay1, by1), 0.0)
    inter = iw * ih
    ua = jnp.maximum(area_a + area_b - inter, 1e-8)
    iou = inter / ua

    iou_max = jnp.max(iou, axis=0, keepdims=True)
    iota_m = lax.broadcasted_iota(jnp.int32, (M, BA), 0)
    amax = jnp.min(jnp.where(iou == iou_max, iota_m, M), axis=0, keepdims=True)
    oh_m = iota_m == amax

    def pick(col):
        return jnp.sum(jnp.where(oh_m, col, 0.0), axis=0, keepdims=True)

    gx1 = pick(bx1)
    gy1 = pick(by1)
    gx2 = pick(bx2)
    gy2 = pick(by2)

    pos = (iou_max >= 0.5) & valid
    contrib = ((iou_max >= 0.5) | (iou_max < 0.4)) & valid
    posf = pos.astype(jnp.float32)
    npos = jnp.sum(posf)

    neg = (0.75 * (x * x)) * (-jnp.log(1.0 - x))
    s_neg = jax.lax.dot_general(jnp.ones((1, C), jnp.float32), neg,
                                (((1,), (0,)), ((), ())),
                                preferred_element_type=jnp.float32)
    blab = annb[:, 4:5]
    glab = pick(blab)
    lab_i = glab.astype(jnp.int32)
    iota_c = lax.broadcasted_iota(jnp.int32, (C, BA), 0)
    x_sel = jnp.sum(jnp.where(iota_c == lab_i, x, 0.0), axis=0, keepdims=True)
    pos_term = (0.25 * (1.0 - x_sel) * (1.0 - x_sel)) * (-jnp.log(x_sel))
    neg_sel = (0.75 * (x_sel * x_sel)) * (-jnp.log(1.0 - x_sel))
    row_cls = (jnp.where(contrib, s_neg, 0.0)
               + jnp.where(pos, pos_term - neg_sel, 0.0))
    cls_s = jnp.sum(row_cls)

    gt_w = gx2 - gx1
    gt_h = gy2 - gy1
    gcx = gx1 + 0.5 * gt_w
    gcy = gy1 + 0.5 * gt_h
    gt_w = jnp.maximum(gt_w, 1.0)
    gt_h = jnp.maximum(gt_h, 1.0)
    tdx = ((gcx - acx) / aw_s) / 0.1
    tdy = ((gcy - acy) / ah_s) / 0.1
    tdw = jnp.log(gt_w / aw_s) / 0.2
    tdh = jnp.log(gt_h / ah_s) / 0.2

    def smooth_l1(t, c):
        d = jnp.abs(t - reg_ref[0, c:c + 1, :])
        return jnp.where(d <= 1.0 / 9.0, 0.5 * 9.0 * (d * d), d - 0.5 / 9.0)

    rl = smooth_l1(tdx, 0) + smooth_l1(tdy, 1) + smooth_l1(tdw, 2) + smooth_l1(tdh, 3)
    reg_s = jnp.sum(jnp.where(pos, rl, 0.0))

    @pl.when(i == 0)
    def _init():
        out_ref[0, 0, 0] = 0.0
        out_ref[0, 0, 1] = 0.0
        out_ref[0, 0, 2] = 0.0
        out_ref[0, 0, 3] = 0.0

    out_ref[0, 0, 0] += cls_s
    out_ref[0, 0, 1] += reg_s
    out_ref[0, 0, 2] += npos


@jax.jit
def kernel(classifications, regressions, anchors, annotations):
    B, A, C = classifications.shape
    M = annotations.shape[1]
    nblk = -(-A // _BA)

    cls_t = classifications.transpose(0, 2, 1)
    reg_t = regressions.transpose(0, 2, 1)
    anch_t = anchors[0].T
    nvalid = jnp.full((1,), A, dtype=jnp.int32)

    out = pl.pallas_call(
        _focal_body,
        grid=(B, nblk),
        in_specs=[
            pl.BlockSpec((1, C, _BA), lambda j, i: (j, 0, i)),
            pl.BlockSpec((1, 4, _BA), lambda j, i: (j, 0, i)),
            pl.BlockSpec((4, _BA), lambda j, i: (0, i)),
            pl.BlockSpec((1, M, 5), lambda j, i: (j, 0, 0)),
            pl.BlockSpec(memory_space=pltpu.SMEM),
        ],
        out_specs=pl.BlockSpec((1, 1, 4), lambda j, i: (j, 0, 0),
                               memory_space=pltpu.SMEM),
        out_shape=jax.ShapeDtypeStruct((B, 1, 4), jnp.float32),
    )(cls_t, reg_t, anch_t, annotations, nvalid)

    cls_sum = out[:, 0, 0]
    reg_sum = out[:, 0, 1]
    npos = out[:, 0, 2]
    cls_loss = jnp.mean(cls_sum / jnp.maximum(npos, 1.0)).reshape(1)
    reg_loss = jnp.mean(reg_sum / jnp.maximum(npos * 4.0, 1.0)).reshape(1)
    return cls_loss, reg_loss


# speedup vs baseline: 1.0188x; 1.0188x over previous
import jax
import jax.numpy as jnp
from jax import lax
from jax.experimental import pallas as pl
from jax.experimental.pallas import tpu as pltpu

_BA = 10240


def _focal_body(cls_ref, reg_ref, anch_ref, ann_ref, nvalid_ref, out_ref):
    i = pl.program_id(1)

    x = jnp.clip(cls_ref[0], 1e-4, 1.0 - 1e-4)
    C, BA = x.shape
    nvalid = nvalid_ref[0]
    valid = (lax.broadcasted_iota(jnp.int32, (1, BA), 1) + i * BA) < nvalid
    x = jnp.where(valid, x, 0.5)
    annb = ann_ref[0]
    M = annb.shape[0]
    bx1 = annb[:, 0:1]
    by1 = annb[:, 1:2]
    bx2 = annb[:, 2:3]
    by2 = annb[:, 3:4]

    ax1 = jnp.where(valid, anch_ref[0:1, :], 0.0)
    ay1 = jnp.where(valid, anch_ref[1:2, :], 0.0)
    ax2 = jnp.where(valid, anch_ref[2:3, :], 16.0)
    ay2 = jnp.where(valid, anch_ref[3:4, :], 16.0)
    aw = ax2 - ax1
    ah = ay2 - ay1
    acx = ax1 + 0.5 * aw
    acy = ay1 + 0.5 * ah
    aw_s = jnp.maximum(aw, 1e-3)
    ah_s = jnp.maximum(ah, 1e-3)

    area_a = aw * ah
    area_b = (bx2 - bx1) * (by2 - by1)
    iw = jnp.maximum(jnp.minimum(ax2, bx2) - jnp.maximum(ax1, bx1), 0.0)
    ih = jnp.maximum(jnp.minimum(ay2, by2) - jnp.maximum(ay1, by1), 0.0)
    inter = iw * ih
    ua = jnp.maximum(area_a + area_b - inter, 1e-8)
    iou = inter / ua

    iou_max = jnp.max(iou, axis=0, keepdims=True)
    iota_m = lax.broadcasted_iota(jnp.int32, (M, BA), 0)
    amax = jnp.min(jnp.where(iou == iou_max, iota_m, M), axis=0, keepdims=True)
    oh_m = iota_m == amax

    def pick(col):
        return jnp.sum(jnp.where(oh_m, col, 0.0), axis=0, keepdims=True)

    gx1 = pick(bx1)
    gy1 = pick(by1)
    gx2 = pick(bx2)
    gy2 = pick(by2)

    pos = (iou_max >= 0.5) & valid
    contrib = ((iou_max >= 0.5) | (iou_max < 0.4)) & valid
    posf = pos.astype(jnp.float32)
    npos = jnp.sum(posf)

    neg = (0.75 * (x * x)) * (-jnp.log(1.0 - x))
    s_neg = jax.lax.dot_general(jnp.ones((1, C), jnp.float32), neg,
                                (((1,), (0,)), ((), ())),
                                preferred_element_type=jnp.float32)
    blab = annb[:, 4:5]
    glab = pick(blab)
    lab_i = glab.astype(jnp.int32)
    iota_c = lax.broadcasted_iota(jnp.int32, (C, BA), 0)
    x_sel = jnp.sum(jnp.where(iota_c == lab_i, x, 0.0), axis=0, keepdims=True)
    pos_term = (0.25 * (1.0 - x_sel) * (1.0 - x_sel)) * (-jnp.log(x_sel))
    neg_sel = (0.75 * (x_sel * x_sel)) * (-jnp.log(1.0 - x_sel))
    row_cls = (jnp.where(contrib, s_neg, 0.0)
               + jnp.where(pos, pos_term - neg_sel, 0.0))
    cls_s = jnp.sum(row_cls)

    gt_w = gx2 - gx1
    gt_h = gy2 - gy1
    gcx = gx1 + 0.5 * gt_w
    gcy = gy1 + 0.5 * gt_h
    gt_w = jnp.maximum(gt_w, 1.0)
    gt_h = jnp.maximum(gt_h, 1.0)
    tdx = ((gcx - acx) / aw_s) / 0.1
    tdy = ((gcy - acy) / ah_s) / 0.1
    tdw = jnp.log(gt_w / aw_s) / 0.2
    tdh = jnp.log(gt_h / ah_s) / 0.2

    def smooth_l1(t, c):
        d = jnp.abs(t - reg_ref[0, c:c + 1, :])
        return jnp.where(d <= 1.0 / 9.0, 0.5 * 9.0 * (d * d), d - 0.5 / 9.0)

    rl = smooth_l1(tdx, 0) + smooth_l1(tdy, 1) + smooth_l1(tdw, 2) + smooth_l1(tdh, 3)
    reg_s = jnp.sum(jnp.where(pos, rl, 0.0))

    @pl.when(i == 0)
    def _init():
        out_ref[0, 0, 0] = 0.0
        out_ref[0, 0, 1] = 0.0
        out_ref[0, 0, 2] = 0.0
        out_ref[0, 0, 3] = 0.0

    out_ref[0, 0, 0] += cls_s
    out_ref[0, 0, 1] += reg_s
    out_ref[0, 0, 2] += npos


@jax.jit
def kernel(classifications, regressions, anchors, annotations):
    B, A, C = classifications.shape
    M = annotations.shape[1]
    nblk = -(-A // _BA)

    cls_t = classifications.transpose(0, 2, 1)
    reg_t = regressions.transpose(0, 2, 1)
    anch_t = anchors[0].T
    nvalid = jnp.full((1,), A, dtype=jnp.int32)

    out = pl.pallas_call(
        _focal_body,
        grid=(B, nblk),
        in_specs=[
            pl.BlockSpec((1, C, _BA), lambda j, i: (j, 0, i)),
            pl.BlockSpec((1, 4, _BA), lambda j, i: (j, 0, i)),
            pl.BlockSpec((4, _BA), lambda j, i: (0, i)),
            pl.BlockSpec((1, M, 5), lambda j, i: (j, 0, 0)),
            pl.BlockSpec(memory_space=pltpu.SMEM),
        ],
        out_specs=pl.BlockSpec((1, 1, 4), lambda j, i: (j, 0, 0),
                               memory_space=pltpu.SMEM),
        out_shape=jax.ShapeDtypeStruct((B, 1, 4), jnp.float32),
    )(cls_t, reg_t, anch_t, annotations, nvalid)

    cls_sum = out[:, 0, 0]
    reg_sum = out[:, 0, 1]
    npos = out[:, 0, 2]
    cls_loss = jnp.mean(cls_sum / jnp.maximum(npos, 1.0)).reshape(1)
    reg_loss = jnp.mean(reg_sum / jnp.maximum(npos * 4.0, 1.0)).reshape(1)
    return cls_loss, reg_loss
